# baseline (speedup 1.0000x reference)
import functools

import jax
import jax.numpy as jnp
from jax import lax
from jax.experimental import pallas as pl
from jax.experimental.pallas import tpu as pltpu
from jax.experimental.pallas import tpu_sc as plsc

N = 10000
E = 320000
D = 128
H = 128
L = 3

NC = 2
NS = 16
NW = NC * NS
CH = 128
ROWS = 2560
EPAD = ROWS * CH
RPW = ROWS // NW
NP = 10240

ET = 2048
NT = 2000


def _silu(v):
    return v * jax.nn.sigmoid(v)


def _sc_gather(AX, BX, dstg, srcg):
    mesh = plsc.VectorSubcoreMesh(core_axis_name="c", subcore_axis_name="s")

    @functools.partial(
        pl.kernel,
        out_type=[
            jax.ShapeDtypeStruct((EPAD, H), jnp.int32),
            jax.ShapeDtypeStruct((EPAD, H), jnp.int32),
        ],
        mesh=mesh,
        scratch_types=[
            pltpu.VMEM((RPW, CH), jnp.int32),
            pltpu.VMEM((RPW, CH), jnp.int32),
            pltpu.VMEM((CH, H), jnp.int32),
            pltpu.VMEM((CH, H), jnp.int32),
            pltpu.VMEM((CH, H), jnp.int32),
            pltpu.VMEM((CH, H), jnp.int32),
            pltpu.SemaphoreType.DMA,
            pltpu.SemaphoreType.DMA,
            pltpu.SemaphoreType.DMA,
            pltpu.SemaphoreType.DMA,
        ],
    )
    def k(ax_hbm, bx_hbm, d_hbm, s_hbm, ga_out, gb_out,
          dv, sv, ga0, gb0, ga1, gb1, sg0, sg1, so0, so1):
        wid = lax.axis_index("s") * NC + lax.axis_index("c")
        base = wid * RPW
        gas = (ga0, ga1)
        gbs = (gb0, gb1)
        sgs = (sg0, sg1)
        sos = (so0, so1)

        pltpu.sync_copy(d_hbm.at[pl.ds(base, RPW)], dv)
        pltpu.sync_copy(s_hbm.at[pl.ds(base, RPW)], sv)

        def stage(b, kk):
            pltpu.async_copy(ax_hbm.at[dv.at[kk]], gas[b], sgs[b])
            pltpu.async_copy(bx_hbm.at[sv.at[kk]], gbs[b], sgs[b])

        def wait_g(b, kk):
            pltpu.make_async_copy(ax_hbm.at[dv.at[kk]], gas[b], sgs[b]).wait()
            pltpu.make_async_copy(bx_hbm.at[sv.at[kk]], gbs[b], sgs[b]).wait()

        def flush(b, kk):
            e0 = (base + kk) * CH
            pltpu.async_copy(gas[b], ga_out.at[pl.ds(e0, CH)], sos[b])
            pltpu.async_copy(gbs[b], gb_out.at[pl.ds(e0, CH)], sos[b])

        def wait_o(b, kk):
            e0 = (base + kk) * CH
            pltpu.make_async_copy(gas[b], ga_out.at[pl.ds(e0, CH)], sos[b]).wait()
            pltpu.make_async_copy(gbs[b], gb_out.at[pl.ds(e0, CH)], sos[b]).wait()

        stage(0, 0)
        stage(1, 1)

        def body(g, carry):
            k0 = 2 * g
            wait_g(0, k0)
            flush(0, k0)
            wait_g(1, k0 + 1)
            flush(1, k0 + 1)
            wait_o(0, k0)
            stage(0, k0 + 2)
            wait_o(1, k0 + 1)
            stage(1, k0 + 3)
            return carry

        lax.fori_loop(0, RPW // 2 - 1, body, 0)

        kl = RPW - 2
        wait_g(0, kl)
        flush(0, kl)
        wait_g(1, kl + 1)
        flush(1, kl + 1)
        wait_o(0, kl)
        wait_o(1, kl + 1)

    return k(AX, BX, dstg, srcg)


def _sc_scatter(m, dsts):
    mesh = plsc.VectorSubcoreMesh(core_axis_name="c", subcore_axis_name="s")
    ZR = NP // NS
    ZC = ZR // CH

    @functools.partial(
        pl.kernel,
        out_type=jax.ShapeDtypeStruct((NC, NP, H), jnp.float32),
        mesh=mesh,
        scratch_types=[
            pltpu.VMEM((RPW, CH), jnp.int32),
            pltpu.VMEM((CH, H), jnp.float32),
            pltpu.VMEM((CH, H), jnp.float32),
            pltpu.VMEM_SHARED((NP, H), jnp.float32),
            pltpu.SemaphoreType.DMA,
            pltpu.SemaphoreType.DMA,
        ],
    )
    def k(m_hbm, d_hbm, am_out, dv, mv0, mv1, accm, si0, si1):
        cid = lax.axis_index("c")
        sid = lax.axis_index("s")
        wid = sid * NC + cid
        base = wid * RPW
        mvs = (mv0, mv1)
        sis = (si0, si1)

        def zrow(j, carry):
            for kk in range(H // 16):
                mv0[j, pl.ds(kk * 16, 16)] = jnp.zeros((16,), jnp.float32)
            return carry
        lax.fori_loop(0, CH, zrow, 0)
        for z in range(ZC):
            pltpu.sync_copy(mv0, accm.at[pl.ds(sid * ZR + z * CH, CH)])
        plsc.subcore_barrier()

        pltpu.sync_copy(d_hbm.at[pl.ds(base, RPW)], dv)

        def stage(b, kk):
            e0 = (base + kk) * CH
            pltpu.async_copy(m_hbm.at[pl.ds(e0, CH)], mvs[b], sis[b])

        def wait_in(b, kk):
            e0 = (base + kk) * CH
            pltpu.make_async_copy(m_hbm.at[pl.ds(e0, CH)], mvs[b], sis[b]).wait()

        stage(0, 0)
        stage(1, 1)

        def body(g, carry):
            k0 = 2 * g
            wait_in(0, k0)
            pltpu.sync_copy(mv0, accm.at[dv.at[k0]], add=True)
            stage(0, k0 + 2)
            wait_in(1, k0 + 1)
            pltpu.sync_copy(mv1, accm.at[dv.at[k0 + 1]], add=True)
            stage(1, k0 + 3)
            return carry

        lax.fori_loop(0, RPW // 2 - 1, body, 0)

        kl = RPW - 2
        wait_in(0, kl)
        pltpu.sync_copy(mv0, accm.at[dv.at[kl]], add=True)
        wait_in(1, kl + 1)
        pltpu.sync_copy(mv1, accm.at[dv.at[kl + 1]], add=True)
        plsc.subcore_barrier()

        for z in range(ZC):
            r0 = sid * ZR + z * CH
            pltpu.sync_copy(accm.at[pl.ds(r0, CH)], mv0)
            pltpu.sync_copy(mv0, am_out.at[cid, pl.ds(r0, CH)])

    return k(m, dsts)


def _unpack_planes(v):
    lo = lax.bitcast_convert_type(v.astype(jnp.int16),
                                  jnp.bfloat16).astype(jnp.float32)
    hi = lax.bitcast_convert_type((v >> 16).astype(jnp.int16),
                                  jnp.bfloat16).astype(jnp.float32)
    return lo, hi


def _edge_body(gxa_ref, gxb_ref, wr_ref, b1_ref, w2_ref, b2_ref,
               c1_ref, cb1_ref, c2_ref, cb2_ref, m_ref, t_ref):
    la, ha = _unpack_planes(gxa_ref[...])
    lb, hb = _unpack_planes(gxb_ref[...])
    HW = H // 2
    g_lo = la[:, 0:HW] + lb[:, 0:HW]
    g_hi = ha[:, 0:HW] + hb[:, 0:HW]
    d_lo = la[:, HW:H] - lb[:, HW:H]
    d_hi = ha[:, HW:H] - hb[:, HW:H]
    r2 = (jnp.sum(d_lo * d_lo, axis=1, keepdims=True)
          + jnp.sum(d_hi * d_hi, axis=1, keepdims=True))
    wr = wr_ref[...]
    b1 = b1_ref[...]
    m1_lo = _silu(g_lo + r2 * wr[:, 0:HW] + b1[:, 0:HW])
    m1_hi = _silu(g_hi + r2 * wr[:, HW:H] + b1[:, HW:H])
    m2 = _silu(jnp.dot(m1_lo, w2_ref[0:HW, :], preferred_element_type=jnp.float32)
               + jnp.dot(m1_hi, w2_ref[HW:H, :], preferred_element_type=jnp.float32)
               + b2_ref[...])
    u = _silu(jnp.dot(m2, c1_ref[...], preferred_element_type=jnp.float32)
              + cb1_ref[...])
    w = jnp.dot(u, c2_ref[...], preferred_element_type=jnp.float32) + cb2_ref[...]
    m_ref[...] = m2
    ii = lax.broadcasted_iota(jnp.int32, (1, HW), 1)
    t_ref[:, 0:HW] = d_lo * w
    t_ref[:, HW:H] = jnp.where(ii == 1, 1.0, d_hi * w)


def _edge_pipeline(GXA, GXB, wr, b1, W2, b2, C1, cb1, C2, cb2):
    grid = (EPAD // ET,)
    full = lambda shape: pl.BlockSpec(shape, lambda i: (0, 0))
    return pl.pallas_call(
        _edge_body,
        grid=grid,
        in_specs=[
            pl.BlockSpec((ET, H), lambda i: (i, 0)),
            pl.BlockSpec((ET, H), lambda i: (i, 0)),
            full((1, H)), full((1, H)), full((H, H)), full((1, H)),
            full((H, H)), full((1, H)), full((H, 1)), full((1, 1)),
        ],
        out_specs=[
            pl.BlockSpec((ET, H), lambda i: (i, 0)),
            pl.BlockSpec((ET, H), lambda i: (i, 0)),
        ],
        out_shape=[
            jax.ShapeDtypeStruct((EPAD, H), jnp.float32),
            jax.ShapeDtypeStruct((EPAD, H), jnp.float32),
        ],
    )(GXA, GXB, wr, b1, W2, b2, C1, cb1, C2, cb2)


def _node_body(h_ref, am0_ref, am1_ref, ax0_ref, ax1_ref, xp_ref,
               wa_ref, wb_ref, nb1_ref, w2_ref, nb2_ref,
               h_out_ref, x_out_ref):
    h = h_ref[...]
    agg = am0_ref[...] + am1_ref[...]
    hn = _silu(jnp.dot(h, wa_ref[...], preferred_element_type=jnp.float32)
               + jnp.dot(agg, wb_ref[...], preferred_element_type=jnp.float32)
               + nb1_ref[...])
    hn = jnp.dot(hn, w2_ref[...], preferred_element_type=jnp.float32) + nb2_ref[...]
    h_out_ref[...] = h + hn
    axs = ax0_ref[...] + ax1_ref[...]
    cnt = jnp.maximum(axs[:, 65:66], 1.0)
    xup = jnp.concatenate(
        [axs[:, 0:1], axs[:, 64:65], axs[:, 1:2],
         jnp.zeros((axs.shape[0], H - 3), jnp.float32)], axis=1)
    x_out_ref[...] = xp_ref[...] + xup / cnt


def _node_pipeline(h, am0, am1, ax0, ax1, xp, Wa, Wb, nb1, W2, nb2):
    grid = (N // NT,)
    full = lambda shape: pl.BlockSpec(shape, lambda i: (0, 0))
    return pl.pallas_call(
        _node_body,
        grid=grid,
        in_specs=[
            pl.BlockSpec((NT, H), lambda i: (i, 0)),
            pl.BlockSpec((NT, H), lambda i: (i, 0)),
            pl.BlockSpec((NT, H), lambda i: (i, 0)),
            pl.BlockSpec((NT, H), lambda i: (i, 0)),
            pl.BlockSpec((NT, H), lambda i: (i, 0)),
            pl.BlockSpec((NT, H), lambda i: (i, 0)),
            full((H, H)), full((H, H)), full((1, H)), full((H, D)), full((1, D)),
        ],
        out_specs=[
            pl.BlockSpec((NT, D), lambda i: (i, 0)),
            pl.BlockSpec((NT, H), lambda i: (i, 0)),
        ],
        out_shape=[
            jax.ShapeDtypeStruct((N, D), jnp.float32),
            jax.ShapeDtypeStruct((N, H), jnp.float32),
        ],
    )(h, am0, am1, ax0, ax1, xp, Wa, Wb, nb1, W2, nb2)


def _proj_body(h_ref, xp_ref, w_ref, ax_ref, bx_ref):
    h = h_ref[...]
    xpb = xp_ref[...].astype(jnp.bfloat16)
    ax_ref[:, 0:H] = jnp.dot(h, w_ref[0:D, :],
                             preferred_element_type=jnp.float32).astype(jnp.bfloat16)
    ax_ref[:, H:2 * H] = xpb
    bx_ref[:, 0:H] = jnp.dot(h, w_ref[D:2 * D, :],
                             preferred_element_type=jnp.float32).astype(jnp.bfloat16)
    bx_ref[:, H:2 * H] = xpb


def _proj_pipeline(h, xp, W1d):
    grid = (N // NT,)
    return pl.pallas_call(
        _proj_body,
        grid=grid,
        in_specs=[
            pl.BlockSpec((NT, D), lambda i: (i, 0)),
            pl.BlockSpec((NT, H), lambda i: (i, 0)),
            pl.BlockSpec((2 * D, H), lambda i: (0, 0)),
        ],
        out_specs=[
            pl.BlockSpec((NT, 2 * H), lambda i: (i, 0)),
            pl.BlockSpec((NT, 2 * H), lambda i: (i, 0)),
        ],
        out_shape=[
            jax.ShapeDtypeStruct((N, 2 * H), jnp.bfloat16),
            jax.ShapeDtypeStruct((N, 2 * H), jnp.bfloat16),
        ],
    )(h, xp, W1d)


def _fc_body(h_ref, w_ref, b_ref, y_ref):
    y_ref[...] = jnp.dot(h_ref[...], w_ref[...],
                         preferred_element_type=jnp.float32) + b_ref[...]


def _fc_pipeline(h, fc_W, fc_b):
    grid = (N // NT,)
    return pl.pallas_call(
        _fc_body,
        grid=grid,
        in_specs=[
            pl.BlockSpec((NT, D), lambda i: (i, 0)),
            pl.BlockSpec((D, 128), lambda i: (0, 0)),
            pl.BlockSpec((1, 128), lambda i: (0, 0)),
        ],
        out_specs=pl.BlockSpec((NT, 128), lambda i: (i, 0)),
        out_shape=jax.ShapeDtypeStruct((N, 128), jnp.float32),
    )(h, fc_W, fc_b[None, :])


def kernel(h, x, edge_index, edge_W1, edge_b1, edge_W2, edge_b2,
           coord_W1, coord_b1, coord_W2, coord_b2,
           node_W1, node_b1, node_W2, node_b2, fc_W, fc_b):
    src = edge_index[0]
    dst = edge_index[1]
    padg = jnp.zeros((EPAD - E,), jnp.int32)
    pads = jnp.full((EPAD - E,), NP - 1, jnp.int32)
    dstg = jnp.concatenate([dst, padg]).reshape(ROWS, CH)
    srcg = jnp.concatenate([src, padg]).reshape(ROWS, CH)
    dsts = jnp.concatenate([dst, pads]).reshape(ROWS, CH)

    xp = jnp.pad(x, ((0, 0), (0, H - 3)))

    perm = jnp.concatenate([jnp.arange(0, H, 2, dtype=jnp.int32),
                            jnp.arange(1, H, 2, dtype=jnp.int32)])

    for l in range(L):
        AX, BX = _proj_pipeline(h, xp, edge_W1[l])
        AXi = lax.bitcast_convert_type(AX.reshape(N, H, 2), jnp.int32)
        BXi = lax.bitcast_convert_type(BX.reshape(N, H, 2), jnp.int32)
        wr = jnp.take(edge_W1[l, 2 * D], perm, axis=0)[None, :]
        b1 = jnp.take(edge_b1[l], perm, axis=0)[None, :]
        W2 = jnp.take(edge_W2[l], perm, axis=0)

        GXA, GXB = _sc_gather(AXi, BXi, dstg, srcg)

        m, t = _edge_pipeline(
            GXA, GXB, wr, b1, W2,
            edge_b2[l][None, :], coord_W1[l], coord_b1[l][None, :],
            coord_W2[l], coord_b2[l][None, :])

        am = _sc_scatter(m, dsts)
        ax = _sc_scatter(t, dsts)

        h, xp = _node_pipeline(
            h, am[0, :N], am[1, :N], ax[0, :N], ax[1, :N], xp,
            node_W1[l, 0:D, :], node_W1[l, D:, :], node_b1[l][None, :],
            node_W2[l], node_b2[l][None, :])

    return _fc_pipeline(h, fc_W, fc_b)
